# Initial kernel scaffold; baseline (speedup 1.0000x reference)
import jax
import jax.numpy as jnp
from jax.experimental import pallas as pl


def kernel(x, edge_index, w_out1, w_root1, w_out2, w_root2, w_out3, w_root3, w_out4, w_root4):
    raise NotImplementedError("write your pallas kernel here")


# speedup vs baseline: 6.7638x; 6.7638x over previous
import functools

import jax
import jax.numpy as jnp
from jax import lax
from jax.experimental import pallas as pl
from jax.experimental.pallas import tpu as pltpu
from jax.experimental.pallas import tpu_sc as plsc

N = 10000
E = 320000
NC = 2
NS = 16
CH = 128
RPT = N // NS


def _make_segsum_edge_split(d):
    ept = E // (NC * NS)
    nfull = ept // CH
    tail = ept - nfull * CH

    mesh = plsc.VectorSubcoreMesh(core_axis_name="c", subcore_axis_name="s", num_cores=NC, num_subcores=NS)

    @functools.partial(
        pl.kernel,
        out_type=jax.ShapeDtypeStruct((NC, N, d), jnp.float32),
        mesh=mesh,
        scratch_types=[
            pltpu.VMEM((CH,), jnp.int32),
            pltpu.VMEM((CH,), jnp.int32),
            pltpu.VMEM((CH, d), jnp.float32),
            pltpu.VMEM((tail,), jnp.int32),
            pltpu.VMEM((tail,), jnp.int32),
            pltpu.VMEM((tail, d), jnp.float32),
            pltpu.VMEM_SHARED((N, d), jnp.float32),
            pltpu.SemaphoreType.DMA,
        ],
        compiler_params=pltpu.CompilerParams(use_tc_tiling_on_sc=False),
    )
    def seg(table, src, dst, zeros, out, sidx, didx, rows, tsidx, tdidx,
            trows, acc, sem):
        c = lax.axis_index("c")
        s = lax.axis_index("s")
        r0 = s * RPT
        pltpu.sync_copy(zeros.at[pl.ds(r0, RPT)], acc.at[pl.ds(r0, RPT)])
        plsc.subcore_barrier()

        ebase = (c * NS + s) * ept

        def body(j, carry):
            base = ebase + j * CH
            pltpu.sync_copy(src.at[pl.ds(base, CH)], sidx)
            pltpu.sync_copy(dst.at[pl.ds(base, CH)], didx)
            pltpu.async_copy(table.at[sidx], rows, sem).wait()
            pltpu.sync_copy(rows, acc.at[didx], add=True)
            return carry

        lax.fori_loop(0, nfull, body, 0)

        if tail:
            baset = ebase + nfull * CH
            pltpu.sync_copy(src.at[pl.ds(baset, tail)], tsidx)
            pltpu.sync_copy(dst.at[pl.ds(baset, tail)], tdidx)
            pltpu.async_copy(table.at[tsidx], trows, sem).wait()
            pltpu.sync_copy(trows, acc.at[tdidx], add=True)

        plsc.subcore_barrier()
        pltpu.sync_copy(acc.at[pl.ds(r0, RPT)], out.at[c].at[pl.ds(r0, RPT)])

    return seg


def _make_segsum_chan_split(d):
    ept = E // NS
    nfull = ept // CH
    tail = ept - nfull * CH

    mesh = plsc.VectorSubcoreMesh(core_axis_name="c", subcore_axis_name="s", num_cores=NC, num_subcores=NS)

    @functools.partial(
        pl.kernel,
        out_type=jax.ShapeDtypeStruct((NC, N, d), jnp.float32),
        mesh=mesh,
        scratch_types=[
            pltpu.VMEM((CH,), jnp.int32),
            pltpu.VMEM((CH,), jnp.int32),
            pltpu.VMEM((CH, d), jnp.float32),
            pltpu.VMEM((tail,), jnp.int32),
            pltpu.VMEM((tail,), jnp.int32),
            pltpu.VMEM((tail, d), jnp.float32),
            pltpu.VMEM_SHARED((N, d), jnp.float32),
            pltpu.SemaphoreType.DMA,
        ],
        compiler_params=pltpu.CompilerParams(use_tc_tiling_on_sc=False),
    )
    def seg(table, src, dst, zeros, out, sidx, didx, rows, tsidx, tdidx,
            trows, acc, sem):
        c = lax.axis_index("c")
        s = lax.axis_index("s")
        r0 = s * RPT
        pltpu.sync_copy(zeros.at[pl.ds(r0, RPT)], acc.at[pl.ds(r0, RPT)])
        plsc.subcore_barrier()

        ebase = s * ept

        def body(j, carry):
            base = ebase + j * CH
            pltpu.sync_copy(src.at[pl.ds(base, CH)], sidx)
            pltpu.sync_copy(dst.at[pl.ds(base, CH)], didx)
            pltpu.async_copy(table.at[c].at[sidx], rows, sem).wait()
            pltpu.sync_copy(rows, acc.at[didx], add=True)
            return carry

        lax.fori_loop(0, nfull, body, 0)

        if tail:
            baset = ebase + nfull * CH
            pltpu.sync_copy(src.at[pl.ds(baset, tail)], tsidx)
            pltpu.sync_copy(dst.at[pl.ds(baset, tail)], tdidx)
            pltpu.async_copy(table.at[c].at[tsidx], trows, sem).wait()
            pltpu.sync_copy(trows, acc.at[tdidx], add=True)

        plsc.subcore_barrier()
        pltpu.sync_copy(acc.at[pl.ds(r0, RPT)], out.at[c].at[pl.ds(r0, RPT)])

    return seg


BM = 400


def _elu(z, alpha=1.0):
    return jnp.where(z > 0, z, alpha * (jnp.exp(jnp.minimum(z, 0.0)) - 1.0))


def _l1_body(p_ref, x_ref, wo_ref, wr_ref, h_ref, dv_ref):
    p = p_ref[0] + p_ref[1]
    deg = p[:, 128:129]
    dv = 1.0 / jnp.maximum(deg, 1.0)
    agg = p[:, :128] * dv
    z = (jnp.dot(agg, wo_ref[...], preferred_element_type=jnp.float32)
         + jnp.dot(x_ref[...], wr_ref[...], preferred_element_type=jnp.float32))
    h = _elu(z)
    h_ref[0] = h[:, :128]
    h_ref[1] = h[:, 128:]
    dv_ref[...] = jnp.broadcast_to(dv, (BM, 128))


def _tc_l1(p1, x, wo, wr):
    return pl.pallas_call(
        _l1_body,
        grid=(N // BM,),
        in_specs=[
            pl.BlockSpec((NC, BM, 144), lambda i: (0, i, 0)),
            pl.BlockSpec((BM, 128), lambda i: (i, 0)),
            pl.BlockSpec((128, 256), lambda i: (0, 0)),
            pl.BlockSpec((128, 256), lambda i: (0, 0)),
        ],
        out_specs=[
            pl.BlockSpec((NC, BM, 128), lambda i: (0, i, 0)),
            pl.BlockSpec((BM, 128), lambda i: (i, 0)),
        ],
        out_shape=[
            jax.ShapeDtypeStruct((NC, N, 128), jnp.float32),
            jax.ShapeDtypeStruct((N, 128), jnp.float32),
        ],
    )(p1, x, wo, wr)


def _l23_body(a_ref, h_ref, dv_ref, wo_ref, wr_ref, o_ref):
    dv = dv_ref[...]
    wo = wo_ref[...]
    wr = wr_ref[...]
    z = (jnp.dot(a_ref[0] * dv, wo[:128], preferred_element_type=jnp.float32)
         + jnp.dot(a_ref[1] * dv, wo[128:], preferred_element_type=jnp.float32)
         + jnp.dot(h_ref[0], wr[:128], preferred_element_type=jnp.float32)
         + jnp.dot(h_ref[1], wr[128:], preferred_element_type=jnp.float32))
    h = _elu(z)
    o_ref[0] = h[:, :128]
    o_ref[1] = h[:, 128:]


def _tc_l2(a, hs, dv, wo, wr):
    return pl.pallas_call(
        _l23_body,
        grid=(N // BM,),
        in_specs=[
            pl.BlockSpec((NC, BM, 128), lambda i: (0, i, 0)),
            pl.BlockSpec((NC, BM, 128), lambda i: (0, i, 0)),
            pl.BlockSpec((BM, 128), lambda i: (i, 0)),
            pl.BlockSpec((256, 256), lambda i: (0, 0)),
            pl.BlockSpec((256, 256), lambda i: (0, 0)),
        ],
        out_specs=pl.BlockSpec((NC, BM, 128), lambda i: (0, i, 0)),
        out_shape=jax.ShapeDtypeStruct((NC, N, 128), jnp.float32),
    )(a, hs, dv, wo, wr)


def _l3_body(a_ref, h_ref, dv_ref, wo_ref, wr_ref, wo4_ref, h3_ref, y_ref):
    dv = dv_ref[...]
    wo = wo_ref[...]
    wr = wr_ref[...]
    z = (jnp.dot(a_ref[0] * dv, wo[:128], preferred_element_type=jnp.float32)
         + jnp.dot(a_ref[1] * dv, wo[128:], preferred_element_type=jnp.float32)
         + jnp.dot(h_ref[0], wr[:128], preferred_element_type=jnp.float32)
         + jnp.dot(h_ref[1], wr[128:], preferred_element_type=jnp.float32))
    h3 = _elu(z)
    h3_ref[...] = h3
    y_ref[...] = jnp.dot(h3, wo4_ref[...], preferred_element_type=jnp.float32)


def _tc_l3(a, hs, dv, wo, wr, wo4):
    return pl.pallas_call(
        _l3_body,
        grid=(N // BM,),
        in_specs=[
            pl.BlockSpec((NC, BM, 128), lambda i: (0, i, 0)),
            pl.BlockSpec((NC, BM, 128), lambda i: (0, i, 0)),
            pl.BlockSpec((BM, 128), lambda i: (i, 0)),
            pl.BlockSpec((256, 256), lambda i: (0, 0)),
            pl.BlockSpec((256, 256), lambda i: (0, 0)),
            pl.BlockSpec((256, 128), lambda i: (0, 0)),
        ],
        out_specs=[
            pl.BlockSpec((BM, 256), lambda i: (i, 0)),
            pl.BlockSpec((BM, 128), lambda i: (i, 0)),
        ],
        out_shape=[
            jax.ShapeDtypeStruct((N, 256), jnp.float32),
            jax.ShapeDtypeStruct((N, 128), jnp.float32),
        ],
    )(a, hs, dv, wo, wr, wo4)


def _l4_body(p_ref, h3_ref, dv_ref, wr_ref, o_ref):
    agg = (p_ref[0] + p_ref[1]) * dv_ref[...]
    z = agg + jnp.dot(h3_ref[...], wr_ref[...],
                      preferred_element_type=jnp.float32)
    o_ref[...] = _elu(z, alpha=128.0)


def _tc_l4(p, h3, dv, wr):
    return pl.pallas_call(
        _l4_body,
        grid=(N // BM,),
        in_specs=[
            pl.BlockSpec((NC, BM, 128), lambda i: (0, i, 0)),
            pl.BlockSpec((BM, 256), lambda i: (i, 0)),
            pl.BlockSpec((BM, 128), lambda i: (i, 0)),
            pl.BlockSpec((256, 128), lambda i: (0, 0)),
        ],
        out_specs=pl.BlockSpec((BM, 128), lambda i: (i, 0)),
        out_shape=jax.ShapeDtypeStruct((N, 128), jnp.float32),
    )(p, h3, dv, wr)


_seg_edge_144 = _make_segsum_edge_split(144)
_seg_edge_128 = _make_segsum_edge_split(128)
_seg_chan_128 = _make_segsum_chan_split(128)


@jax.jit
def kernel(x, edge_index, w_out1, w_root1, w_out2, w_root2, w_out3, w_root3,
           w_out4, w_root4):
    src = edge_index[0]
    dst = edge_index[1]
    xt = jnp.concatenate(
        [x, jnp.ones((N, 1), jnp.float32), jnp.zeros((N, 15), jnp.float32)],
        axis=1)
    z144 = jnp.zeros((N, 144), jnp.float32)
    z128 = jnp.zeros((N, 128), jnp.float32)

    p1 = _seg_edge_144(xt, src, dst, z144)
    h1s, dv = _tc_l1(p1, x, w_out1, w_root1)
    a2 = _seg_chan_128(h1s, src, dst, z128)
    h2s = _tc_l2(a2, h1s, dv, w_out2, w_root2)
    a3 = _seg_chan_128(h2s, src, dst, z128)
    h3, y = _tc_l3(a3, h2s, dv, w_out3, w_root3, w_out4)
    p4 = _seg_edge_128(y, src, dst, z128)
    return _tc_l4(p4, h3, dv, w_root4)


# speedup vs baseline: 9.4253x; 1.3935x over previous
import functools

import jax
import jax.numpy as jnp
from jax import lax
from jax.experimental import pallas as pl
from jax.experimental.pallas import tpu as pltpu
from jax.experimental.pallas import tpu_sc as plsc

N = 10000
E = 320000
NC = 2
NS = 16
CH = 128
CG = 64
RPT = N // NS


NCHUNK = E // CH
NCHPAD = 2512


def _make_segsum(npass):
    q, r = divmod(NCHUNK, NS)
    maxch = q + 1

    mesh = plsc.VectorSubcoreMesh(core_axis_name="c", subcore_axis_name="s",
                                  num_cores=NC, num_subcores=NS)

    @functools.partial(
        pl.kernel,
        out_type=jax.ShapeDtypeStruct((NC * npass, N, CG), jnp.float32),
        mesh=mesh,
        scratch_types=[
            pltpu.VMEM((maxch * CH,), jnp.int32),
            pltpu.VMEM((CH,), jnp.int32),
            pltpu.VMEM((CH,), jnp.int32),
            pltpu.VMEM((CH, CG), jnp.float32),
            pltpu.VMEM((CH, CG), jnp.float32),
            pltpu.VMEM_SHARED((N, CG), jnp.float32),
            pltpu.VMEM_SHARED((N, CG), jnp.float32),
            pltpu.SemaphoreType.DMA,
            pltpu.SemaphoreType.DMA,
            pltpu.SemaphoreType.DMA,
            pltpu.SemaphoreType.DMA,
        ],
        compiler_params=pltpu.CompilerParams(use_tc_tiling_on_sc=False),
    )
    def seg(table, src1d, dst1d, zeros, out, sidx, didx0, didx1, rows0, rows1,
            tbl, acc, gsem0, gsem1, dsem0, dsem1):
        c = lax.axis_index("c")
        s = lax.axis_index("s")
        r0 = s * RPT

        chunk0 = s * q + lax.min(s, r)
        nch = q + jnp.where(s < r, 1, 0)
        e0 = chunk0 * CH
        pltpu.sync_copy(src1d.at[pl.ds(e0, maxch * CH)], sidx)

        rows = (rows0, rows1)
        didx = (didx0, didx1)
        gsem = (gsem0, gsem1)
        dsem = (dsem0, dsem1)

        for p in range(npass):
            g = c * npass + p
            pltpu.sync_copy(table.at[g].at[pl.ds(r0, RPT)],
                            tbl.at[pl.ds(r0, RPT)])
            pltpu.sync_copy(zeros, acc.at[pl.ds(r0, RPT)])
            plsc.subcore_barrier()

            pltpu.async_copy(tbl.at[sidx.at[pl.ds(0, CH)]], rows0, gsem0)
            pltpu.async_copy(dst1d.at[pl.ds(e0, CH)], didx0, dsem0)

            def step(b, j):
                nb = 1 - b

                @pl.when(j + 1 < nch)
                def _prefetch():
                    o = (j + 1) * CH
                    pltpu.async_copy(tbl.at[sidx.at[pl.ds(o, CH)]], rows[nb],
                                     gsem[nb])
                    pltpu.async_copy(dst1d.at[pl.ds(e0 + o, CH)], didx[nb],
                                     dsem[nb])

                o = j * CH
                pltpu.make_async_copy(tbl.at[sidx.at[pl.ds(o, CH)]], rows[b],
                                      gsem[b]).wait()
                pltpu.make_async_copy(dst1d.at[pl.ds(e0 + o, CH)], didx[b],
                                      dsem[b]).wait()
                pltpu.sync_copy(rows[b], acc.at[didx[b]], add=True)

            def body(j, carry):
                @pl.when(j % 2 == 0)
                def _even():
                    step(0, j)

                @pl.when(j % 2 == 1)
                def _odd():
                    step(1, j)

                return carry

            lax.fori_loop(0, nch, body, 0)

            plsc.subcore_barrier()
            pltpu.sync_copy(acc.at[pl.ds(r0, RPT)],
                            out.at[g].at[pl.ds(r0, RPT)])

    return seg


DW = 16


def _make_deg():
    q, r = divmod(NCHUNK, NC * NS)
    mesh = plsc.VectorSubcoreMesh(core_axis_name="c", subcore_axis_name="s",
                                  num_cores=NC, num_subcores=NS)

    @functools.partial(
        pl.kernel,
        out_type=jax.ShapeDtypeStruct((NC, N, DW), jnp.float32),
        mesh=mesh,
        scratch_types=[
            pltpu.VMEM((CH,), jnp.int32),
            pltpu.VMEM((CH,), jnp.int32),
            pltpu.VMEM((CH, DW), jnp.float32),
            pltpu.VMEM_SHARED((N, DW), jnp.float32),
            pltpu.SemaphoreType.DMA,
            pltpu.SemaphoreType.DMA,
        ],
        compiler_params=pltpu.CompilerParams(use_tc_tiling_on_sc=False),
    )
    def deg(ones_blk, dst1d, zeros, out, didx0, didx1, rones, acc,
            dsem0, dsem1):
        c = lax.axis_index("c")
        s = lax.axis_index("s")
        r0 = s * RPT
        pltpu.sync_copy(zeros, acc.at[pl.ds(r0, RPT)])
        pltpu.sync_copy(ones_blk, rones)

        w = c * NS + s
        chunk0 = w * q + lax.min(w, r)
        nch = q + jnp.where(w < r, 1, 0)
        e0 = chunk0 * CH
        plsc.subcore_barrier()

        didx = (didx0, didx1)
        dsem = (dsem0, dsem1)
        pltpu.async_copy(dst1d.at[pl.ds(e0, CH)], didx0, dsem0)

        def step(b, j):
            nb = 1 - b

            @pl.when(j + 1 < nch)
            def _prefetch():
                pltpu.async_copy(dst1d.at[pl.ds(e0 + (j + 1) * CH, CH)],
                                 didx[nb], dsem[nb])

            pltpu.make_async_copy(dst1d.at[pl.ds(e0 + j * CH, CH)], didx[b],
                                  dsem[b]).wait()
            pltpu.sync_copy(rones, acc.at[didx[b]], add=True)

        def body(j, carry):
            @pl.when(j % 2 == 0)
            def _even():
                step(0, j)

            @pl.when(j % 2 == 1)
            def _odd():
                step(1, j)

            return carry

        lax.fori_loop(0, nch, body, 0)

        plsc.subcore_barrier()
        pltpu.sync_copy(acc.at[pl.ds(r0, RPT)], out.at[c].at[pl.ds(r0, RPT)])

    return deg


BM = 400


def _elu(z, alpha=1.0):
    return jnp.where(z > 0, z, alpha * (jnp.exp(jnp.minimum(z, 0.0)) - 1.0))


def _l1_body(p_ref, pd_ref, x_ref, wo_ref, wr_ref, h_ref, dv_ref):
    deg = pd_ref[0][:, :1] + pd_ref[1][:, :1]
    dv = 1.0 / jnp.maximum(deg, 1.0)
    z = (jnp.dot(p_ref[0] * dv, wo_ref[...][:CG],
                 preferred_element_type=jnp.float32)
         + jnp.dot(p_ref[1] * dv, wo_ref[...][CG:],
                   preferred_element_type=jnp.float32)
         + jnp.dot(x_ref[...], wr_ref[...], preferred_element_type=jnp.float32))
    h = _elu(z)
    for g in range(4):
        h_ref[g] = h[:, g * CG:(g + 1) * CG]
    dv_ref[...] = jnp.broadcast_to(dv, (BM, 128))


def _tc_l1(p1, pd, x, wo, wr):
    return pl.pallas_call(
        _l1_body,
        grid=(N // BM,),
        in_specs=[
            pl.BlockSpec((2, BM, CG), lambda i: (0, i, 0)),
            pl.BlockSpec((NC, BM, DW), lambda i: (0, i, 0)),
            pl.BlockSpec((BM, 128), lambda i: (i, 0)),
            pl.BlockSpec((128, 256), lambda i: (0, 0)),
            pl.BlockSpec((128, 256), lambda i: (0, 0)),
        ],
        out_specs=[
            pl.BlockSpec((4, BM, CG), lambda i: (0, i, 0)),
            pl.BlockSpec((BM, 128), lambda i: (i, 0)),
        ],
        out_shape=[
            jax.ShapeDtypeStruct((4, N, CG), jnp.float32),
            jax.ShapeDtypeStruct((N, 128), jnp.float32),
        ],
    )(p1, pd, x, wo, wr)


def _l23_body(a_ref, h_ref, dv_ref, wo_ref, wr_ref, o_ref):
    dv = dv_ref[...][:, :CG]
    wo = wo_ref[...]
    wr = wr_ref[...]
    z = 0.0
    for g in range(4):
        z = (z + jnp.dot(a_ref[g] * dv, wo[g * CG:(g + 1) * CG],
                         preferred_element_type=jnp.float32)
             + jnp.dot(h_ref[g], wr[g * CG:(g + 1) * CG],
                       preferred_element_type=jnp.float32))
    h = _elu(z)
    for g in range(4):
        o_ref[g] = h[:, g * CG:(g + 1) * CG]


def _tc_l2(a, hs, dv, wo, wr):
    return pl.pallas_call(
        _l23_body,
        grid=(N // BM,),
        in_specs=[
            pl.BlockSpec((4, BM, CG), lambda i: (0, i, 0)),
            pl.BlockSpec((4, BM, CG), lambda i: (0, i, 0)),
            pl.BlockSpec((BM, 128), lambda i: (i, 0)),
            pl.BlockSpec((256, 256), lambda i: (0, 0)),
            pl.BlockSpec((256, 256), lambda i: (0, 0)),
        ],
        out_specs=pl.BlockSpec((4, BM, CG), lambda i: (0, i, 0)),
        out_shape=jax.ShapeDtypeStruct((4, N, CG), jnp.float32),
    )(a, hs, dv, wo, wr)


def _l3_body(a_ref, h_ref, dv_ref, wo_ref, wr_ref, wo4_ref, h3_ref, y_ref):
    dv = dv_ref[...][:, :CG]
    wo = wo_ref[...]
    wr = wr_ref[...]
    z = 0.0
    for g in range(4):
        z = (z + jnp.dot(a_ref[g] * dv, wo[g * CG:(g + 1) * CG],
                         preferred_element_type=jnp.float32)
             + jnp.dot(h_ref[g], wr[g * CG:(g + 1) * CG],
                       preferred_element_type=jnp.float32))
    h3 = _elu(z)
    h3_ref[...] = h3
    y = jnp.dot(h3, wo4_ref[...], preferred_element_type=jnp.float32)
    y_ref[0] = y[:, :CG]
    y_ref[1] = y[:, CG:]


def _tc_l3(a, hs, dv, wo, wr, wo4):
    return pl.pallas_call(
        _l3_body,
        grid=(N // BM,),
        in_specs=[
            pl.BlockSpec((4, BM, CG), lambda i: (0, i, 0)),
            pl.BlockSpec((4, BM, CG), lambda i: (0, i, 0)),
            pl.BlockSpec((BM, 128), lambda i: (i, 0)),
            pl.BlockSpec((256, 256), lambda i: (0, 0)),
            pl.BlockSpec((256, 256), lambda i: (0, 0)),
            pl.BlockSpec((256, 128), lambda i: (0, 0)),
        ],
        out_specs=[
            pl.BlockSpec((BM, 256), lambda i: (i, 0)),
            pl.BlockSpec((2, BM, CG), lambda i: (0, i, 0)),
        ],
        out_shape=[
            jax.ShapeDtypeStruct((N, 256), jnp.float32),
            jax.ShapeDtypeStruct((2, N, CG), jnp.float32),
        ],
    )(a, hs, dv, wo, wr, wo4)


def _l4_body(p_ref, h3_ref, dv_ref, wr_ref, o_ref):
    dv = dv_ref[...][:, :CG]
    agg = jnp.concatenate([p_ref[0] * dv, p_ref[1] * dv], axis=1)
    z = agg + jnp.dot(h3_ref[...], wr_ref[...],
                      preferred_element_type=jnp.float32)
    o_ref[...] = _elu(z, alpha=128.0)


def _tc_l4(p, h3, dv, wr):
    return pl.pallas_call(
        _l4_body,
        grid=(N // BM,),
        in_specs=[
            pl.BlockSpec((2, BM, CG), lambda i: (0, i, 0)),
            pl.BlockSpec((BM, 256), lambda i: (i, 0)),
            pl.BlockSpec((BM, 128), lambda i: (i, 0)),
            pl.BlockSpec((256, 128), lambda i: (0, 0)),
        ],
        out_specs=pl.BlockSpec((BM, 128), lambda i: (i, 0)),
        out_shape=jax.ShapeDtypeStruct((N, 128), jnp.float32),
    )(p, h3, dv, wr)


_seg_1pass = _make_segsum(1)
_seg_2pass = _make_segsum(2)
_deg_kernel = _make_deg()


@jax.jit
def kernel(x, edge_index, w_out1, w_root1, w_out2, w_root2, w_out3, w_root3,
           w_out4, w_root4):
    npad = (NCHPAD - NCHUNK) * CH
    src1d = jnp.pad(edge_index[0], (0, npad))
    dst1d = jnp.pad(edge_index[1], (0, npad))
    zcg = jnp.zeros((RPT, CG), jnp.float32)
    zdw = jnp.zeros((RPT, DW), jnp.float32)
    ones_blk = jnp.ones((CH, DW), jnp.float32)

    xq = x.reshape(N, 2, CG).transpose(1, 0, 2)

    pd = _deg_kernel(ones_blk, dst1d, zdw)
    p1 = _seg_1pass(xq, src1d, dst1d, zcg)
    h1s, dv = _tc_l1(p1, pd, x, w_out1, w_root1)
    a2 = _seg_2pass(h1s, src1d, dst1d, zcg)
    h2s = _tc_l2(a2, h1s, dv, w_out2, w_root2)
    a3 = _seg_2pass(h2s, src1d, dst1d, zcg)
    h3, y = _tc_l3(a3, h2s, dv, w_out3, w_root3, w_out4)
    p4 = _seg_1pass(y, src1d, dst1d, zcg)
    return _tc_l4(p4, h3, dv, w_root4)


# speedup vs baseline: 10.9156x; 1.1581x over previous
import functools

import jax
import jax.numpy as jnp
from jax import lax
from jax.experimental import pallas as pl
from jax.experimental.pallas import tpu as pltpu
from jax.experimental.pallas import tpu_sc as plsc

N = 10000
E = 320000
NC = 2
NS = 16
CH = 128
CG = 64
RPT = N // NS
NCHUNK = E // CH


def _make_segsum(npass):
    q, r = divmod(NCHUNK, NS)
    maxch = q + 1

    mesh = plsc.VectorSubcoreMesh(core_axis_name="c", subcore_axis_name="s",
                                  num_cores=NC, num_subcores=NS)
    out_shape = (N, 128) if npass == 1 else (NC, N, 128)

    @functools.partial(
        pl.kernel,
        out_type=jax.ShapeDtypeStruct(out_shape, jnp.float32),
        mesh=mesh,
        scratch_types=[
            pltpu.VMEM((maxch * CH,), jnp.int32),
            pltpu.VMEM((CH,), jnp.int32),
            pltpu.VMEM((CH,), jnp.int32),
            pltpu.VMEM((CH, CG), jnp.float32),
            pltpu.VMEM((CH, CG), jnp.float32),
            pltpu.VMEM_SHARED((N, CG), jnp.float32),
            pltpu.VMEM_SHARED((N, CG), jnp.float32),
            pltpu.SemaphoreType.DMA,
            pltpu.SemaphoreType.DMA,
            pltpu.SemaphoreType.DMA,
            pltpu.SemaphoreType.DMA,
            pltpu.SemaphoreType.DMA,
            pltpu.SemaphoreType.DMA,
        ],
        compiler_params=pltpu.CompilerParams(use_tc_tiling_on_sc=False),
    )
    def seg(table, ei1d, zeros, out, sidx, didx0, didx1, rows0, rows1,
            tbl, acc, gsem0, gsem1, dsem0, dsem1, ssem0, ssem1):
        c = lax.axis_index("c")
        s = lax.axis_index("s")
        r0 = s * RPT

        chunk0 = s * q + lax.min(s, r)
        nch = q + jnp.where(s < r, 1, 0)
        e0 = chunk0 * CH
        pltpu.sync_copy(ei1d.at[pl.ds(e0, maxch * CH)], sidx)

        rows = (rows0, rows1)
        didx = (didx0, didx1)
        gsem = (gsem0, gsem1)
        dsem = (dsem0, dsem1)
        ssem = (ssem0, ssem1)

        for p in range(npass):
            if npass == 1:
                src2d = table
                dst2d = out
                col = c * CG
            else:
                src2d = table.at[c]
                dst2d = out.at[c]
                col = p * CG
            pltpu.sync_copy(src2d.at[pl.ds(r0, RPT), pl.ds(col, CG)],
                            tbl.at[pl.ds(r0, RPT)])
            pltpu.sync_copy(zeros, acc.at[pl.ds(r0, RPT)])
            plsc.subcore_barrier()

            pltpu.async_copy(tbl.at[sidx.at[pl.ds(0, CH)]], rows0, gsem0)
            pltpu.async_copy(ei1d.at[pl.ds(E + e0, CH)], didx0, dsem0)

            def step(b, j):
                nb = 1 - b

                @pl.when(j + 1 < nch)
                def _prefetch():
                    o = (j + 1) * CH

                    @pl.when(j >= 1)
                    def _drain_prev():
                        pltpu.make_async_copy(rows[nb], acc.at[didx[nb]],
                                              ssem[nb]).wait()

                    pltpu.async_copy(tbl.at[sidx.at[pl.ds(o, CH)]], rows[nb],
                                     gsem[nb])
                    pltpu.async_copy(ei1d.at[pl.ds(E + e0 + o, CH)], didx[nb],
                                     dsem[nb])

                o = j * CH
                pltpu.make_async_copy(tbl.at[sidx.at[pl.ds(o, CH)]], rows[b],
                                      gsem[b]).wait()
                pltpu.make_async_copy(ei1d.at[pl.ds(E + e0 + o, CH)], didx[b],
                                      dsem[b]).wait()
                pltpu.async_copy(rows[b], acc.at[didx[b]], ssem[b], add=True)

            def body(j, carry):
                @pl.when(j % 2 == 0)
                def _even():
                    step(0, j)

                @pl.when(j % 2 == 1)
                def _odd():
                    step(1, j)

                return carry

            lax.fori_loop(0, nch, body, 0)

            pltpu.make_async_copy(rows0, acc.at[didx0], ssem0).wait()
            pltpu.make_async_copy(rows1, acc.at[didx1], ssem1).wait()

            plsc.subcore_barrier()
            pltpu.sync_copy(acc.at[pl.ds(r0, RPT)],
                            dst2d.at[pl.ds(r0, RPT), pl.ds(col, CG)])

    return seg


DW = 16


def _make_deg():
    q, r = divmod(NCHUNK, NC * NS)
    mesh = plsc.VectorSubcoreMesh(core_axis_name="c", subcore_axis_name="s",
                                  num_cores=NC, num_subcores=NS)

    @functools.partial(
        pl.kernel,
        out_type=jax.ShapeDtypeStruct((NC, N, DW), jnp.float32),
        mesh=mesh,
        scratch_types=[
            pltpu.VMEM((CH,), jnp.int32),
            pltpu.VMEM((CH,), jnp.int32),
            pltpu.VMEM((CH, DW), jnp.float32),
            pltpu.VMEM_SHARED((N, DW), jnp.float32),
            pltpu.SemaphoreType.DMA,
            pltpu.SemaphoreType.DMA,
        ],
        compiler_params=pltpu.CompilerParams(use_tc_tiling_on_sc=False),
    )
    def deg(ones_blk, ei1d, zeros, out, didx0, didx1, rones, acc,
            dsem0, dsem1):
        c = lax.axis_index("c")
        s = lax.axis_index("s")
        r0 = s * RPT
        pltpu.sync_copy(zeros, acc.at[pl.ds(r0, RPT)])
        pltpu.sync_copy(ones_blk, rones)

        w = c * NS + s
        chunk0 = w * q + lax.min(w, r)
        nch = q + jnp.where(w < r, 1, 0)
        e0 = chunk0 * CH
        plsc.subcore_barrier()

        didx = (didx0, didx1)
        dsem = (dsem0, dsem1)
        pltpu.async_copy(ei1d.at[pl.ds(E + e0, CH)], didx0, dsem0)

        def step(b, j):
            nb = 1 - b

            @pl.when(j + 1 < nch)
            def _prefetch():
                pltpu.async_copy(ei1d.at[pl.ds(E + e0 + (j + 1) * CH, CH)],
                                 didx[nb], dsem[nb])

            pltpu.make_async_copy(ei1d.at[pl.ds(E + e0 + j * CH, CH)], didx[b],
                                  dsem[b]).wait()
            pltpu.sync_copy(rones, acc.at[didx[b]], add=True)

        def body(j, carry):
            @pl.when(j % 2 == 0)
            def _even():
                step(0, j)

            @pl.when(j % 2 == 1)
            def _odd():
                step(1, j)

            return carry

        lax.fori_loop(0, nch, body, 0)

        plsc.subcore_barrier()
        pltpu.sync_copy(acc.at[pl.ds(r0, RPT)], out.at[c].at[pl.ds(r0, RPT)])

    return deg


BM = 400


def _elu(z, alpha=1.0):
    return jnp.where(z > 0, z, alpha * (jnp.exp(jnp.minimum(z, 0.0)) - 1.0))


def _l1_body(p_ref, pd_ref, x_ref, wo_ref, wr_ref, h_ref, dv_ref):
    deg = pd_ref[0][:, :1] + pd_ref[1][:, :1]
    dv = 1.0 / jnp.maximum(deg, 1.0)
    z = (jnp.dot(p_ref[...] * dv, wo_ref[...],
                 preferred_element_type=jnp.float32)
         + jnp.dot(x_ref[...], wr_ref[...], preferred_element_type=jnp.float32))
    h = _elu(z)
    h_ref[0] = h[:, :128]
    h_ref[1] = h[:, 128:]
    dv_ref[...] = jnp.broadcast_to(dv, (BM, 128))


def _tc_l1(p1, pd, x, wo, wr):
    return pl.pallas_call(
        _l1_body,
        grid=(N // BM,),
        in_specs=[
            pl.BlockSpec((BM, 128), lambda i: (i, 0)),
            pl.BlockSpec((NC, BM, DW), lambda i: (0, i, 0)),
            pl.BlockSpec((BM, 128), lambda i: (i, 0)),
            pl.BlockSpec((128, 256), lambda i: (0, 0)),
            pl.BlockSpec((128, 256), lambda i: (0, 0)),
        ],
        out_specs=[
            pl.BlockSpec((NC, BM, 128), lambda i: (0, i, 0)),
            pl.BlockSpec((BM, 128), lambda i: (i, 0)),
        ],
        out_shape=[
            jax.ShapeDtypeStruct((NC, N, 128), jnp.float32),
            jax.ShapeDtypeStruct((N, 128), jnp.float32),
        ],
    )(p1, pd, x, wo, wr)


def _l23_body(a_ref, h_ref, dv_ref, wo_ref, wr_ref, o_ref):
    dv = dv_ref[...]
    wo = wo_ref[...]
    wr = wr_ref[...]
    z = (jnp.dot(a_ref[0] * dv, wo[:128], preferred_element_type=jnp.float32)
         + jnp.dot(a_ref[1] * dv, wo[128:], preferred_element_type=jnp.float32)
         + jnp.dot(h_ref[0], wr[:128], preferred_element_type=jnp.float32)
         + jnp.dot(h_ref[1], wr[128:], preferred_element_type=jnp.float32))
    h = _elu(z)
    o_ref[0] = h[:, :128]
    o_ref[1] = h[:, 128:]


def _tc_l2(a, hs, dv, wo, wr):
    return pl.pallas_call(
        _l23_body,
        grid=(N // BM,),
        in_specs=[
            pl.BlockSpec((NC, BM, 128), lambda i: (0, i, 0)),
            pl.BlockSpec((NC, BM, 128), lambda i: (0, i, 0)),
            pl.BlockSpec((BM, 128), lambda i: (i, 0)),
            pl.BlockSpec((256, 256), lambda i: (0, 0)),
            pl.BlockSpec((256, 256), lambda i: (0, 0)),
        ],
        out_specs=pl.BlockSpec((NC, BM, 128), lambda i: (0, i, 0)),
        out_shape=jax.ShapeDtypeStruct((NC, N, 128), jnp.float32),
    )(a, hs, dv, wo, wr)


def _l3_body(a_ref, h_ref, dv_ref, wo_ref, wr_ref, wo4_ref, h3_ref, y_ref):
    dv = dv_ref[...]
    wo = wo_ref[...]
    wr = wr_ref[...]
    z = (jnp.dot(a_ref[0] * dv, wo[:128], preferred_element_type=jnp.float32)
         + jnp.dot(a_ref[1] * dv, wo[128:], preferred_element_type=jnp.float32)
         + jnp.dot(h_ref[0], wr[:128], preferred_element_type=jnp.float32)
         + jnp.dot(h_ref[1], wr[128:], preferred_element_type=jnp.float32))
    h3 = _elu(z)
    h3_ref[...] = h3
    y_ref[...] = jnp.dot(h3, wo4_ref[...], preferred_element_type=jnp.float32)


def _tc_l3(a, hs, dv, wo, wr, wo4):
    return pl.pallas_call(
        _l3_body,
        grid=(N // BM,),
        in_specs=[
            pl.BlockSpec((NC, BM, 128), lambda i: (0, i, 0)),
            pl.BlockSpec((NC, BM, 128), lambda i: (0, i, 0)),
            pl.BlockSpec((BM, 128), lambda i: (i, 0)),
            pl.BlockSpec((256, 256), lambda i: (0, 0)),
            pl.BlockSpec((256, 256), lambda i: (0, 0)),
            pl.BlockSpec((256, 128), lambda i: (0, 0)),
        ],
        out_specs=[
            pl.BlockSpec((BM, 256), lambda i: (i, 0)),
            pl.BlockSpec((BM, 128), lambda i: (i, 0)),
        ],
        out_shape=[
            jax.ShapeDtypeStruct((N, 256), jnp.float32),
            jax.ShapeDtypeStruct((N, 128), jnp.float32),
        ],
    )(a, hs, dv, wo, wr, wo4)


def _l4_body(p_ref, h3_ref, dv_ref, wr_ref, o_ref):
    agg = p_ref[...] * dv_ref[...]
    z = agg + jnp.dot(h3_ref[...], wr_ref[...],
                      preferred_element_type=jnp.float32)
    o_ref[...] = _elu(z, alpha=128.0)


def _tc_l4(p, h3, dv, wr):
    return pl.pallas_call(
        _l4_body,
        grid=(N // BM,),
        in_specs=[
            pl.BlockSpec((BM, 128), lambda i: (i, 0)),
            pl.BlockSpec((BM, 256), lambda i: (i, 0)),
            pl.BlockSpec((BM, 128), lambda i: (i, 0)),
            pl.BlockSpec((256, 128), lambda i: (0, 0)),
        ],
        out_specs=pl.BlockSpec((BM, 128), lambda i: (i, 0)),
        out_shape=jax.ShapeDtypeStruct((N, 128), jnp.float32),
    )(p, h3, dv, wr)


_seg_1pass = _make_segsum(1)
_seg_2pass = _make_segsum(2)
_deg_kernel = _make_deg()


@jax.jit
def kernel(x, edge_index, w_out1, w_root1, w_out2, w_root2, w_out3, w_root3,
           w_out4, w_root4):
    ei1d = edge_index.reshape(2 * E)
    zcg = jnp.zeros((RPT, CG), jnp.float32)
    zdw = jnp.zeros((RPT, DW), jnp.float32)
    ones_blk = jnp.ones((CH, DW), jnp.float32)

    pd = _deg_kernel(ones_blk, ei1d, zdw)
    p1 = _seg_1pass(x, ei1d, zcg)
    h1s, dv = _tc_l1(p1, pd, x, w_out1, w_root1)
    a2 = _seg_2pass(h1s, ei1d, zcg)
    h2s = _tc_l2(a2, h1s, dv, w_out2, w_root2)
    a3 = _seg_2pass(h2s, ei1d, zcg)
    h3, y = _tc_l3(a3, h2s, dv, w_out3, w_root3, w_out4)
    p4 = _seg_1pass(y, ei1d, zcg)
    return _tc_l4(p4, h3, dv, w_root4)
